# baseline (speedup 1.0000x reference)
import functools

import jax
import jax.numpy as jnp
from jax import lax
from jax.experimental import pallas as pl
from jax.experimental.pallas import tpu as pltpu
from jax.experimental.pallas import tpu_sc as plsc

D = 1024
P_PAD = 72
N_CELL = 16
HI = jax.lax.Precision.HIGHEST

_c11 = (((1,), (1,)), ((), ()))


DC = 256


def _tc_body(hf_ref, hc_ref, cbef_ref, cbec_ref, cbf_ref, cbc_ref,
             wih_ref, whh_ref, winj_ref, bih_ref, bhh_ref,
             idx_ref, cm_ref, inj_ref, sacc, nacc, cnacc):
    c = pl.program_id(0)
    h4 = hf_ref[...][:, :4, :].reshape(N_CELL, D)
    h4c = hc_ref[...][:, :4, :].reshape(N_CELL, DC)
    cbe = cbef_ref[...]
    cbec = cbec_ref[...]
    cb = cbf_ref[...]
    cbc = cbc_ref[...]
    wih = wih_ref[...]
    whh = whh_ref[...]
    bih = bih_ref[...]
    bhh = bhh_ref[...]

    @pl.when(c == 0)
    def _():
        sacc[...] = jnp.zeros_like(sacc)
        nacc[...] = jnp.zeros_like(nacc)
        cnacc[...] = jnp.zeros_like(cnacc)

    gir = lax.dot_general(h4, wih[0], _c11, precision=HI) + bih[0:1, :]
    giz = lax.dot_general(h4, wih[1], _c11, precision=HI) + bih[1:2, :]
    gin = lax.dot_general(h4, wih[2], _c11, precision=HI) + bih[2:3, :]
    ghr = lax.dot_general(cbe, whh[0], _c11, precision=HI) + bhh[0:1, :]
    ghz = lax.dot_general(cbe, whh[1], _c11, precision=HI) + bhh[1:2, :]
    ghn = lax.dot_general(cbe, whh[2], _c11, precision=HI) + bhh[2:3, :]

    r3 = jax.nn.sigmoid(gir[:, None, :] + ghr[None, :, :])
    z3 = jax.nn.sigmoid(giz[:, None, :] + ghz[None, :, :])
    n3 = jnp.tanh(gin[:, None, :] + r3 * ghn[None, :, :])
    new3 = (1.0 - z3) * n3 + z3 * cbec[None, :, :]
    p_iota = lax.broadcasted_iota(jnp.int32, (N_CELL, P_PAD, DC), 1)
    new3 = jnp.where(p_iota == 65, h4c[:, None, :], new3)

    newf = new3.reshape(N_CELL * P_PAD, DC)
    sacc[...] += lax.dot_general(newf, cbc, _c11, precision=HI)
    nacc[...] += jnp.sum(newf * newf, axis=1, keepdims=True)
    ones = jnp.ones((1, DC), jnp.float32)
    cnacc[...] += lax.dot_general(ones, cbc * cbc, _c11, precision=HI)

    inj_ref[0:64, :] = lax.dot_general(cb, winj_ref[...], _c11, precision=HI)
    inj_ref[64:128, :] = jnp.zeros((64, DC), jnp.float32)

    @pl.when(c == D // DC - 1)
    def _():
        score = cnacc[...] - 2.0 * sacc[...]
        idx_ref[...] = jnp.argmin(score, axis=1,
                                  keepdims=True).astype(jnp.int32)
        cm_ref[...] = (nacc[...]
                       + jnp.min(score, axis=1, keepdims=True)) * (1.0 / D)


def _tc_tables(h, cbe, W_ih3, W_hh3, W_inj, bih3, bhh3):
    cb = cbe[1:65]
    nsteps = D // DC
    return pl.pallas_call(
        _tc_body,
        grid=(nsteps,),
        in_specs=[
            pl.BlockSpec((4, 8, D), lambda c: (0, 0, 0)),
            pl.BlockSpec((4, 8, DC), lambda c: (0, 0, c)),
            pl.BlockSpec((P_PAD, D), lambda c: (0, 0)),
            pl.BlockSpec((P_PAD, DC), lambda c: (0, c)),
            pl.BlockSpec((64, D), lambda c: (0, 0)),
            pl.BlockSpec((64, DC), lambda c: (0, c)),
            pl.BlockSpec((3, DC, D), lambda c: (0, c, 0)),
            pl.BlockSpec((3, DC, D), lambda c: (0, c, 0)),
            pl.BlockSpec((DC, D), lambda c: (c, 0)),
            pl.BlockSpec((3, DC), lambda c: (0, c)),
            pl.BlockSpec((3, DC), lambda c: (0, c)),
        ],
        out_specs=[
            pl.BlockSpec((N_CELL * P_PAD, 1), lambda c: (0, 0)),
            pl.BlockSpec((N_CELL * P_PAD, 1), lambda c: (0, 0)),
            pl.BlockSpec((128, DC), lambda c: (0, c)),
        ],
        out_shape=[
            jax.ShapeDtypeStruct((N_CELL * P_PAD, 1), jnp.int32),
            jax.ShapeDtypeStruct((N_CELL * P_PAD, 1), jnp.float32),
            jax.ShapeDtypeStruct((128, D), jnp.float32),
        ],
        scratch_shapes=[
            pltpu.VMEM((N_CELL * P_PAD, 64), jnp.float32),
            pltpu.VMEM((N_CELL * P_PAD, 1), jnp.float32),
            pltpu.VMEM((1, 64), jnp.float32),
        ],
    )(h, h, cbe, cbe, cb, cb, W_ih3, W_hh3, W_inj, bih3, bhh3)


def _sc_fill_body(zrow_hbm, out_hbm, zbuf_v, sem):
    wid = lax.axis_index("s") * 2 + lax.axis_index("c")

    @pl.when(wid < 16)
    def _():
        pltpu.sync_copy(zrow_hbm, zbuf_v)
        wmod = wid % 4
        base = pl.multiple_of((wid // 4) * 2048 + 8 + wmod * 512, 8)
        copies = [
            pltpu.async_copy(zbuf_v, out_hbm.at[pl.ds(base + 16 * j, 16)],
                             sem)
            for j in range(31)
        ]
        for c in copies:
            c.wait()

        @pl.when(wmod < 3)
        def _():
            pltpu.async_copy(zbuf_v, out_hbm.at[pl.ds(base + 496, 16)],
                             sem).wait()

        @pl.when(wmod == 3)
        def _():
            pltpu.async_copy(zbuf_v.at[pl.ds(0, 8)],
                             out_hbm.at[pl.ds(base + 496, 8)], sem).wait()


def _sc_fill(zrow):
    mesh = plsc.VectorSubcoreMesh(core_axis_name="c", subcore_axis_name="s")
    run = functools.partial(
        pl.kernel, _sc_fill_body, mesh=mesh,
        compiler_params=pltpu.CompilerParams(needs_layout_passes=False),
        out_type=jax.ShapeDtypeStruct((4 * 2048, D), jnp.float32),
        scratch_types=[
            pltpu.VMEM((16, D), jnp.float32),
            pltpu.SemaphoreType.DMA,
        ],
    )()
    return run(zrow)


def _sc_body(tags_hbm, idx_hbm, cm_hbm, char_hbm, inj_hbm, head_hbm, avg_hbm,
             tags_v, idx_v, cm_v, char_v, act_v, slots_v, fin_v, rows_v,
             avg_v, sem):
    wid = lax.axis_index("s") * 2 + lax.axis_index("c")

    @pl.when(wid == 0)
    def _():
        pltpu.sync_copy(tags_hbm, tags_v)
        pltpu.sync_copy(idx_hbm, idx_v)
        pltpu.sync_copy(cm_hbm, cm_v)
        pltpu.sync_copy(char_hbm, char_v)
        zeros16 = jnp.zeros((16,), jnp.int32)
        act_v[...] = zeros16
        slots_v[...] = zeros16
        f64 = jnp.full((16,), 64, jnp.int32)
        fin_v[pl.ds(0, 16)] = f64
        fin_v[pl.ds(16, 16)] = f64
        char = char_v[...]
        lane = lax.broadcasted_iota(jnp.int32, (16,), 0)
        m0 = lane == 0

        def step(t, carry):
            tc, nu = carry
            tsp = jnp.full((16,), t, jnp.int32)
            b = plsc.load_gather(tags_v, [tsp])
            s = plsc.load_gather(tags_v, [tsp + 128])
            tok = plsc.load_gather(tags_v, [tsp + 256])
            cell = b * 4 + s
            act = plsc.load_gather(act_v, [b])
            is_char = tok == char
            has_act = act > 0
            slot_b = jnp.where(has_act, (act - 1) & 3, 0)
            p = plsc.load_gather(slots_v, [b * 4 + slot_b])
            flat = jnp.where(is_char, cell * P_PAD + 65, cell * P_PAD + p)
            code = plsc.load_gather(idx_v, [flat])
            cm = plsc.load_gather(cm_v, [flat])
            did = jnp.logical_or(is_char, has_act)
            slot_u = jnp.where(is_char, act & 3, slot_b)
            plsc.store_scatter(slots_v, [b * 4 + slot_u], code + 1,
                               mask=jnp.logical_and(m0, did))
            ich = is_char.astype(jnp.int32)
            plsc.store_scatter(act_v, [b], act + ich, mask=m0)
            tc = tc + jnp.where(did, cm, 0.0)
            nu = nu + jnp.where(did, 1, 0)
            act2 = act + ich
            inj_code = plsc.load_gather(slots_v, [b * 4 + ((act2 - 1) & 3)]) - 1
            plsc.store_scatter(fin_v, [b * 8 + s], inj_code,
                               mask=jnp.logical_and(m0, act2 > 0))
            return tc, nu

        tc, nu = lax.fori_loop(
            0, 128, step,
            (jnp.zeros((16,), jnp.float32), jnp.zeros((16,), jnp.int32)))
        avg_v[...] = tc / jnp.maximum(nu, 1).astype(jnp.float32)
        pltpu.sync_copy(avg_v, avg_hbm)
        pltpu.async_copy(inj_hbm.at[fin_v], rows_v, sem).wait()
        pltpu.sync_copy(rows_v, head_hbm)


def _sc_automaton(tags, idxf, cmf, charv, inj_tab):
    mesh = plsc.VectorSubcoreMesh(core_axis_name="c", subcore_axis_name="s")
    run = functools.partial(
        pl.kernel, _sc_body, mesh=mesh,
        compiler_params=pltpu.CompilerParams(needs_layout_passes=False),
        out_type=[
            jax.ShapeDtypeStruct((32, D), jnp.float32),
            jax.ShapeDtypeStruct((16,), jnp.float32),
        ],
        scratch_types=[
            pltpu.VMEM((512,), jnp.int32),
            pltpu.VMEM((N_CELL * P_PAD,), jnp.int32),
            pltpu.VMEM((N_CELL * P_PAD,), jnp.float32),
            pltpu.VMEM((16,), jnp.int32),
            pltpu.VMEM((16,), jnp.int32),
            pltpu.VMEM((16,), jnp.int32),
            pltpu.VMEM((32,), jnp.int32),
            pltpu.VMEM((32, D), jnp.float32),
            pltpu.VMEM((16,), jnp.float32),
            pltpu.SemaphoreType.DMA,
        ],
    )()
    return run(tags, idxf, cmf, charv, inj_tab)


def kernel(h, tag_positions, char_tag_id, tag_ids_set, codebook, W_ih, W_hh,
           b_ih, b_hh, W_inj):
    B, T, d = h.shape
    cb = codebook.astype(jnp.float32)
    cbe = jnp.zeros((P_PAD, d), jnp.float32).at[1:65].set(cb)
    tags = jnp.zeros((4, 128), jnp.int32).at[:3].set(
        tag_positions.astype(jnp.int32).T).reshape(512)
    charv = jnp.full((16,), char_tag_id, jnp.int32)
    bih3 = b_ih.reshape(3, d).astype(jnp.float32)
    bhh3 = b_hh.reshape(3, d).astype(jnp.float32)

    idxf, cmf, inj_tab = _tc_tables(h.astype(jnp.float32), cbe,
                                    W_ih.reshape(3, d, d).astype(jnp.float32),
                                    W_hh.reshape(3, d, d).astype(jnp.float32),
                                    W_inj.astype(jnp.float32), bih3, bhh3)
    filled = _sc_fill(jnp.zeros((16, d), jnp.float32))
    head, avg = _sc_automaton(tags, idxf.reshape(N_CELL * P_PAD),
                              cmf.reshape(N_CELL * P_PAD), charv, inj_tab)
    injection = filled.reshape(B, T, d).at[:, :8, :].set(
        head.reshape(B, 8, d))
    return injection, avg[0]


# speedup vs baseline: 1.2557x; 1.2557x over previous
import functools

import jax
import jax.numpy as jnp
from jax import lax
from jax.experimental import pallas as pl
from jax.experimental.pallas import tpu as pltpu
from jax.experimental.pallas import tpu_sc as plsc

D = 1024
P_PAD = 72
N_CELL = 16
HI = jax.lax.Precision.HIGHEST

_c11 = (((1,), (1,)), ((), ()))


DC = 256


def _tc_body(hf_ref, hc_ref, cbef_ref, cbec_ref, cbf_ref, cbc_ref,
             wih_ref, whh_ref, winj_ref, bih_ref, bhh_ref,
             idx_ref, cm_ref, inj_ref, zout_ref, sacc, nacc, cnacc):
    zout_ref[...] = jnp.zeros_like(zout_ref)
    c = pl.program_id(0)
    h4 = hf_ref[...][:, :4, :].reshape(N_CELL, D)
    h4c = hc_ref[...][:, :4, :].reshape(N_CELL, DC)
    cbe = cbef_ref[...]
    cbec = cbec_ref[...]
    cb = cbf_ref[...]
    cbc = cbc_ref[...]
    wih = wih_ref[...]
    whh = whh_ref[...]
    bih = bih_ref[...]
    bhh = bhh_ref[...]

    @pl.when(c == 0)
    def _():
        sacc[...] = jnp.zeros_like(sacc)
        nacc[...] = jnp.zeros_like(nacc)
        cnacc[...] = jnp.zeros_like(cnacc)

    gir = lax.dot_general(h4, wih[0], _c11, precision=HI) + bih[0:1, :]
    giz = lax.dot_general(h4, wih[1], _c11, precision=HI) + bih[1:2, :]
    gin = lax.dot_general(h4, wih[2], _c11, precision=HI) + bih[2:3, :]
    ghr = lax.dot_general(cbe, whh[0], _c11, precision=HI) + bhh[0:1, :]
    ghz = lax.dot_general(cbe, whh[1], _c11, precision=HI) + bhh[1:2, :]
    ghn = lax.dot_general(cbe, whh[2], _c11, precision=HI) + bhh[2:3, :]

    r3 = jax.nn.sigmoid(gir[:, None, :] + ghr[None, :, :])
    z3 = jax.nn.sigmoid(giz[:, None, :] + ghz[None, :, :])
    n3 = jnp.tanh(gin[:, None, :] + r3 * ghn[None, :, :])
    new3 = (1.0 - z3) * n3 + z3 * cbec[None, :, :]
    p_iota = lax.broadcasted_iota(jnp.int32, (N_CELL, P_PAD, DC), 1)
    new3 = jnp.where(p_iota == 65, h4c[:, None, :], new3)

    newf = new3.reshape(N_CELL * P_PAD, DC)
    sacc[...] += lax.dot_general(newf, cbc, _c11, precision=HI)
    nacc[...] += jnp.sum(newf * newf, axis=1, keepdims=True)
    ones = jnp.ones((1, DC), jnp.float32)
    cnacc[...] += lax.dot_general(ones, cbc * cbc, _c11, precision=HI)

    inj_ref[0:64, :] = lax.dot_general(cb, winj_ref[...], _c11, precision=HI)
    inj_ref[64:128, :] = jnp.zeros((64, DC), jnp.float32)

    @pl.when(c == D // DC - 1)
    def _():
        score = cnacc[...] - 2.0 * sacc[...]
        idx_ref[...] = jnp.argmin(score, axis=1,
                                  keepdims=True).astype(jnp.int32)
        cm_ref[...] = (nacc[...]
                       + jnp.min(score, axis=1, keepdims=True)) * (1.0 / D)


def _tc_tables(h, cbe, W_ih3, W_hh3, W_inj, bih3, bhh3):
    cb = cbe[1:65]
    nsteps = D // DC
    return pl.pallas_call(
        _tc_body,
        grid=(nsteps,),
        in_specs=[
            pl.BlockSpec((4, 8, D), lambda c: (0, 0, 0)),
            pl.BlockSpec((4, 8, DC), lambda c: (0, 0, c)),
            pl.BlockSpec((P_PAD, D), lambda c: (0, 0)),
            pl.BlockSpec((P_PAD, DC), lambda c: (0, c)),
            pl.BlockSpec((64, D), lambda c: (0, 0)),
            pl.BlockSpec((64, DC), lambda c: (0, c)),
            pl.BlockSpec((3, DC, D), lambda c: (0, c, 0)),
            pl.BlockSpec((3, DC, D), lambda c: (0, c, 0)),
            pl.BlockSpec((DC, D), lambda c: (c, 0)),
            pl.BlockSpec((3, DC), lambda c: (0, c)),
            pl.BlockSpec((3, DC), lambda c: (0, c)),
        ],
        out_specs=[
            pl.BlockSpec((N_CELL * P_PAD, 1), lambda c: (0, 0)),
            pl.BlockSpec((N_CELL * P_PAD, 1), lambda c: (0, 0)),
            pl.BlockSpec((128, DC), lambda c: (0, c)),
            pl.BlockSpec((2048, D), lambda c: (c, 0)),
        ],
        out_shape=[
            jax.ShapeDtypeStruct((N_CELL * P_PAD, 1), jnp.int32),
            jax.ShapeDtypeStruct((N_CELL * P_PAD, 1), jnp.float32),
            jax.ShapeDtypeStruct((128, D), jnp.float32),
            jax.ShapeDtypeStruct((4 * 2048, D), jnp.float32),
        ],
        scratch_shapes=[
            pltpu.VMEM((N_CELL * P_PAD, 64), jnp.float32),
            pltpu.VMEM((N_CELL * P_PAD, 1), jnp.float32),
            pltpu.VMEM((1, 64), jnp.float32),
        ],
    )(h, h, cbe, cbe, cb, cb, W_ih3, W_hh3, W_inj, bih3, bhh3)


def _sc_body(tags_hbm, idx_hbm, cm_hbm, char_hbm, inj_hbm, head_hbm, avg_hbm,
             tags_v, idx_v, cm_v, char_v, act_v, slots_v, fin_v, rows_v,
             avg_v, sem):
    wid = lax.axis_index("s") * 2 + lax.axis_index("c")

    @pl.when(wid == 0)
    def _():
        pltpu.sync_copy(tags_hbm, tags_v)
        pltpu.sync_copy(idx_hbm, idx_v)
        pltpu.sync_copy(cm_hbm, cm_v)
        pltpu.sync_copy(char_hbm, char_v)
        zeros16 = jnp.zeros((16,), jnp.int32)
        act_v[...] = zeros16
        slots_v[...] = zeros16
        f64 = jnp.full((16,), 64, jnp.int32)
        fin_v[pl.ds(0, 16)] = f64
        fin_v[pl.ds(16, 16)] = f64
        char = char_v[...]
        lane = lax.broadcasted_iota(jnp.int32, (16,), 0)
        m0 = lane == 0

        def step(t, carry):
            tc, nu = carry
            tsp = jnp.full((16,), t, jnp.int32)
            b = plsc.load_gather(tags_v, [tsp])
            s = plsc.load_gather(tags_v, [tsp + 128])
            tok = plsc.load_gather(tags_v, [tsp + 256])
            cell = b * 4 + s
            act = plsc.load_gather(act_v, [b])
            is_char = tok == char
            has_act = act > 0
            slot_b = jnp.where(has_act, (act - 1) & 3, 0)
            p = plsc.load_gather(slots_v, [b * 4 + slot_b])
            flat = jnp.where(is_char, cell * P_PAD + 65, cell * P_PAD + p)
            code = plsc.load_gather(idx_v, [flat])
            cm = plsc.load_gather(cm_v, [flat])
            did = jnp.logical_or(is_char, has_act)
            slot_u = jnp.where(is_char, act & 3, slot_b)
            plsc.store_scatter(slots_v, [b * 4 + slot_u], code + 1,
                               mask=jnp.logical_and(m0, did))
            ich = is_char.astype(jnp.int32)
            plsc.store_scatter(act_v, [b], act + ich, mask=m0)
            tc = tc + jnp.where(did, cm, 0.0)
            nu = nu + jnp.where(did, 1, 0)
            act2 = act + ich
            inj_code = plsc.load_gather(slots_v, [b * 4 + ((act2 - 1) & 3)]) - 1
            plsc.store_scatter(fin_v, [b * 8 + s], inj_code,
                               mask=jnp.logical_and(m0, act2 > 0))
            return tc, nu

        tc, nu = lax.fori_loop(
            0, 128, step,
            (jnp.zeros((16,), jnp.float32), jnp.zeros((16,), jnp.int32)))
        avg_v[...] = tc / jnp.maximum(nu, 1).astype(jnp.float32)
        pltpu.sync_copy(avg_v, avg_hbm)
        pltpu.async_copy(inj_hbm.at[fin_v], rows_v, sem).wait()
        pltpu.sync_copy(rows_v, head_hbm)


def _sc_automaton(tags, idxf, cmf, charv, inj_tab):
    mesh = plsc.VectorSubcoreMesh(core_axis_name="c", subcore_axis_name="s")
    run = functools.partial(
        pl.kernel, _sc_body, mesh=mesh,
        compiler_params=pltpu.CompilerParams(needs_layout_passes=False),
        out_type=[
            jax.ShapeDtypeStruct((32, D), jnp.float32),
            jax.ShapeDtypeStruct((16,), jnp.float32),
        ],
        scratch_types=[
            pltpu.VMEM((512,), jnp.int32),
            pltpu.VMEM((N_CELL * P_PAD,), jnp.int32),
            pltpu.VMEM((N_CELL * P_PAD,), jnp.float32),
            pltpu.VMEM((16,), jnp.int32),
            pltpu.VMEM((16,), jnp.int32),
            pltpu.VMEM((16,), jnp.int32),
            pltpu.VMEM((32,), jnp.int32),
            pltpu.VMEM((32, D), jnp.float32),
            pltpu.VMEM((16,), jnp.float32),
            pltpu.SemaphoreType.DMA,
        ],
    )()
    return run(tags, idxf, cmf, charv, inj_tab)


def kernel(h, tag_positions, char_tag_id, tag_ids_set, codebook, W_ih, W_hh,
           b_ih, b_hh, W_inj):
    B, T, d = h.shape
    cb = codebook.astype(jnp.float32)
    cbe = jnp.zeros((P_PAD, d), jnp.float32).at[1:65].set(cb)
    tags = jnp.zeros((4, 128), jnp.int32).at[:3].set(
        tag_positions.astype(jnp.int32).T).reshape(512)
    charv = jnp.full((16,), char_tag_id, jnp.int32)
    bih3 = b_ih.reshape(3, d).astype(jnp.float32)
    bhh3 = b_hh.reshape(3, d).astype(jnp.float32)

    idxf, cmf, inj_tab, zeros_out = _tc_tables(
        h.astype(jnp.float32), cbe,
        W_ih.reshape(3, d, d).astype(jnp.float32),
        W_hh.reshape(3, d, d).astype(jnp.float32),
        W_inj.astype(jnp.float32), bih3, bhh3)
    head, avg = _sc_automaton(tags, idxf.reshape(N_CELL * P_PAD),
                              cmf.reshape(N_CELL * P_PAD), charv, inj_tab)
    injection = zeros_out.reshape(B, T, d).at[:, :8, :].set(
        head.reshape(B, 8, d))
    return injection, avg[0]
